# baseline (speedup 1.0000x reference)
import jax
import jax.numpy as jnp
from jax import lax
from jax.experimental import pallas as pl
from jax.experimental.pallas import tpu as pltpu
from jax.experimental.pallas import tpu_sc as plsc

N_NODES_K = 10000
NPAD = 10240
N_EDGES_K = 320000
NDIM = 128
EDIM = 16
N_SEEDS_K = 100000

NC = 2
NS = 16
NW = NC * NS

EPT1 = N_EDGES_K // NS
CH1 = 400
NCH1 = EPT1 // CH1
GRP1 = CH1 // 16
ACC_W = NPAD * 8

S_CHUNK = 128
S_NFULL = 24
S_TAIL_BIG = 3128 - S_NFULL * S_CHUNK
S_TAIL_SMALL = 3120 - S_NFULL * S_CHUNK
S_SPLIT = 20

_mesh = lambda: plsc.VectorSubcoreMesh(core_axis_name="c", subcore_axis_name="s")


def _seg_body(dst_hbm, ef_hbm, zeros_hbm, sum_out, cnt_out,
              dst_v, ef_v, acc, cnt):
    c = lax.axis_index("c")
    s = lax.axis_index("s")

    for k in range(ACC_W // 8192):
        pltpu.sync_copy(zeros_hbm.at[pl.ds(k * 8192, 8192)],
                        acc.at[pl.ds(k * 8192, 8192)])
    pltpu.sync_copy(zeros_hbm.at[pl.ds(0, NPAD)], cnt)

    lane = jax.lax.broadcasted_iota(jnp.int32, (16,), 0)
    pat01 = lane // 8
    iota8x2 = lane % 8
    ones16 = jnp.full((16,), 1.0, jnp.float32)

    dpat = pat01 * 16 + iota8x2 + c * 8

    def chunk(j, carry):
        e_base = s * EPT1 + j * CH1
        pltpu.sync_copy(dst_hbm.at[pl.ds(e_base, CH1)], dst_v)
        pltpu.sync_copy(ef_hbm.at[s * NCH1 + j], ef_v)

        def group(g, cc):
            dvec = dst_v[pl.ds(g * 16, 16)]

            @pl.when(c == 0)
            def _():
                plsc.addupdate_scatter(cnt, (dvec,), ones16)

            for h in range(8):
                e0 = g * 16 + h * 2
                drep = plsc.load_gather(dst_v, (pat01 + e0,))
                idx = drep * 8 + iota8x2
                data = plsc.load_gather(ef_v, (dpat + e0 * 16,))
                plsc.addupdate_scatter(acc, (idx,), data)
            return cc

        lax.fori_loop(0, GRP1, group, 0)
        return carry

    lax.fori_loop(0, NCH1, chunk, 0)

    out_row = c * NS + s
    pltpu.sync_copy(acc, sum_out.at[out_row])

    @pl.when(c == 0)
    def _():
        pltpu.sync_copy(cnt, cnt_out.at[s])


def _segment_sums(dst_r, ef_r, zeros_in):
    f = pl.kernel(
        _seg_body,
        out_type=[
            jax.ShapeDtypeStruct((NW, ACC_W), jnp.float32),
            jax.ShapeDtypeStruct((NS, NPAD), jnp.float32),
        ],
        mesh=_mesh(),
        compiler_params=pltpu.CompilerParams(needs_layout_passes=False),
        scratch_types=[
            pltpu.VMEM((CH1,), jnp.int32),
            pltpu.VMEM((CH1 * 16,), jnp.float32),
            pltpu.VMEM((ACC_W,), jnp.float32),
            pltpu.VMEM((NPAD,), jnp.float32),
        ],
    )
    return f(dst_r, ef_r, zeros_in)


def _proj_body(sp, cp, wn, bn, w1, w2, be, h_ref, a_ref, b_ref):
    bm = h_ref.shape[0]
    x = sp[...]
    lo = x[:NS].reshape(NS, bm, 8).sum(axis=0)
    hi = x[NS:].reshape(NS, bm, 8).sum(axis=0)
    sums = jnp.concatenate([lo, hi], axis=1)
    cnts = jnp.maximum(cp[...].sum(axis=0), 1.0)[:, None]
    hn = sums / cnts
    h = jnp.maximum(
        lax.dot_general(hn, wn[...], (((1,), (1,)), ((), ())),
                        preferred_element_type=jnp.float32) + bn[...],
        0.0)
    h_ref[...] = h
    a_ref[...] = lax.dot_general(h, w1[...], (((1,), (1,)), ((), ())),
                                 preferred_element_type=jnp.float32) + be[...]
    b_ref[...] = lax.dot_general(h, w2[...], (((1,), (1,)), ((), ())),
                                 preferred_element_type=jnp.float32)


def _projection(sums_p, cnts_p, W_neigh, b_neigh, W1, W2, b_edge):
    BM = 1024
    grid = NPAD // BM
    full2 = lambda shp: pl.BlockSpec(shp, lambda i: (0, 0))
    outp = pl.BlockSpec((BM, NDIM), lambda i: (i, 0))
    return pl.pallas_call(
        _proj_body,
        grid=(grid,),
        in_specs=[
            pl.BlockSpec((NW, BM * 8), lambda i: (0, i)),
            pl.BlockSpec((NS, BM), lambda i: (0, i)),
            full2((NDIM, EDIM)),
            full2((1, NDIM)),
            full2((NDIM, NDIM)),
            full2((NDIM, NDIM)),
            full2((1, NDIM)),
        ],
        out_specs=[outp, outp, outp],
        out_shape=[
            jax.ShapeDtypeStruct((NPAD, NDIM), jnp.float32),
            jax.ShapeDtypeStruct((NPAD, NDIM), jnp.float32),
            jax.ShapeDtypeStruct((NPAD, NDIM), jnp.float32),
        ],
    )(sums_p, cnts_p, W_neigh, b_neigh.reshape(1, NDIM), W1, W2,
      b_edge.reshape(1, NDIM))


def _pair_body(su_hbm, sv_hbm, a_hbm, b_hbm, out_hbm, idxu_v, idxv_v, bufa, bufb):
    c = lax.axis_index("c")
    s = lax.axis_index("s")
    w = s * NC + c
    base = jnp.where(w < S_SPLIT, w * 3128,
                     S_SPLIT * 3128 + (w - S_SPLIT) * 3120)

    pltpu.sync_copy(su_hbm.at[pl.ds(base, 3072)], idxu_v.at[pl.ds(0, 3072)])
    pltpu.sync_copy(sv_hbm.at[pl.ds(base, 3072)], idxv_v.at[pl.ds(0, 3072)])

    @pl.when(w < S_SPLIT)
    def _():
        pltpu.sync_copy(su_hbm.at[pl.ds(base + 3072, S_TAIL_BIG)],
                        idxu_v.at[pl.ds(3072, S_TAIL_BIG)])
        pltpu.sync_copy(sv_hbm.at[pl.ds(base + 3072, S_TAIL_BIG)],
                        idxv_v.at[pl.ds(3072, S_TAIL_BIG)])

    @pl.when(w >= S_SPLIT)
    def _():
        pltpu.sync_copy(su_hbm.at[pl.ds(base + 3072, S_TAIL_SMALL)],
                        idxu_v.at[pl.ds(3072, S_TAIL_SMALL)])
        pltpu.sync_copy(sv_hbm.at[pl.ds(base + 3072, S_TAIL_SMALL)],
                        idxv_v.at[pl.ds(3072, S_TAIL_SMALL)])

    def do_chunk(j, n):
        off = base + j * S_CHUNK
        pltpu.sync_copy(a_hbm.at[idxu_v.at[pl.ds(j * S_CHUNK, n)]], bufa.at[pl.ds(0, n)])
        pltpu.sync_copy(b_hbm.at[idxv_v.at[pl.ds(j * S_CHUNK, n)]], bufb.at[pl.ds(0, n)])

        def addrow(p, cc):
            for rr in range(2):
                r = p * 2 + rr
                for k in range(NDIM // 16):
                    sl = pl.ds(k * 16, 16)
                    bufa[r, sl] = bufa[r, sl] + bufb[r, sl]
            return cc

        lax.fori_loop(0, n // 2, addrow, 0)
        pltpu.sync_copy(bufa.at[pl.ds(0, n)], out_hbm.at[pl.ds(off, n)])

    def chunk(j, carry):
        do_chunk(j, S_CHUNK)
        return carry

    lax.fori_loop(0, S_NFULL, chunk, 0)

    @pl.when(w < S_SPLIT)
    def _():
        do_chunk(S_NFULL, S_TAIL_BIG)

    @pl.when(w >= S_SPLIT)
    def _():
        do_chunk(S_NFULL, S_TAIL_SMALL)


def _pair_gather(su_r, sv_r, A, B):
    f = pl.kernel(
        _pair_body,
        out_type=[jax.ShapeDtypeStruct((N_SEEDS_K, NDIM), jnp.float32)],
        mesh=_mesh(),
        compiler_params=pltpu.CompilerParams(needs_layout_passes=False),
        scratch_types=[
            pltpu.VMEM((3136,), jnp.int32),
            pltpu.VMEM((3136,), jnp.int32),
            pltpu.VMEM((S_CHUNK, NDIM), jnp.float32),
            pltpu.VMEM((S_CHUNK, NDIM), jnp.float32),
        ],
    )
    return f(su_r, sv_r, A, B)[0]


def kernel(nfeats, efeats, edge_index, seeds_u, seeds_v, W_neigh, b_neigh, W_edge, b_edge):
    del nfeats
    dst_r = edge_index[1].astype(jnp.int32)
    ef_r = efeats.reshape(NS * NCH1, CH1 * 16)
    zeros_in = jnp.zeros((ACC_W,), jnp.float32)
    sums_p, cnts_p = _segment_sums(dst_r, ef_r, zeros_in)
    W1 = W_edge[:, :NDIM]
    W2 = W_edge[:, NDIM:]
    h, A, B = _projection(sums_p, cnts_p, W_neigh, b_neigh, W1, W2, b_edge)
    su_r = seeds_u.astype(jnp.int32)
    sv_r = seeds_v.astype(jnp.int32)
    edge = _pair_gather(su_r, sv_r, A, B)
    return (h[:N_NODES_K], edge)


# speedup vs baseline: 1.1680x; 1.1680x over previous
import jax
import jax.numpy as jnp
from jax import lax
from jax.experimental import pallas as pl
from jax.experimental.pallas import tpu as pltpu
from jax.experimental.pallas import tpu_sc as plsc

N_NODES_K = 10000
NPAD = 10240
N_EDGES_K = 320000
NDIM = 128
EDIM = 16
N_SEEDS_K = 100000

NC = 2
NS = 16
NW = NC * NS

EPT1 = N_EDGES_K // NS
CH1 = 400
NCH1 = EPT1 // CH1
GRP1 = CH1 // 16
ACC_W = NPAD * 8

S_CHUNK = 128
S_NFULL = 24
S_TAIL_BIG = 3128 - S_NFULL * S_CHUNK
S_TAIL_SMALL = 3120 - S_NFULL * S_CHUNK
S_SPLIT = 20

_mesh = lambda: plsc.VectorSubcoreMesh(core_axis_name="c", subcore_axis_name="s")


def _seg_body(dst_hbm, ef_hbm, zeros_hbm, sum_out, cnt_out,
              dst_v0, dst_v1, ef_v0, ef_v1, acc, cnt,
              sd0, sd1, se0, se1):
    c = lax.axis_index("c")
    s = lax.axis_index("s")

    for k in range(ACC_W // 8192):
        pltpu.sync_copy(zeros_hbm.at[pl.ds(k * 8192, 8192)],
                        acc.at[pl.ds(k * 8192, 8192)])
    pltpu.sync_copy(zeros_hbm.at[pl.ds(0, NPAD)], cnt)

    lane = jax.lax.broadcasted_iota(jnp.int32, (16,), 0)
    pat01 = lane // 8
    iota8x2 = lane % 8
    ones16 = jnp.full((16,), 1.0, jnp.float32)

    dpat = pat01 * 16 + iota8x2 + c * 8

    bufs = ((dst_v0, ef_v0, sd0, se0), (dst_v1, ef_v1, sd1, se1))

    def start(j, b):
        dv, ev, sd, se = bufs[b]
        e_base = s * EPT1 + j * CH1
        pltpu.async_copy(dst_hbm.at[pl.ds(e_base, CH1)], dv, sd)
        pltpu.async_copy(ef_hbm.at[s * NCH1 + j], ev, se)

    def wait(j, b):
        dv, ev, sd, se = bufs[b]
        e_base = s * EPT1 + j * CH1
        pltpu.make_async_copy(dst_hbm.at[pl.ds(e_base, CH1)], dv, sd).wait()
        pltpu.make_async_copy(ef_hbm.at[s * NCH1 + j], ev, se).wait()

    def process(b):
        dv, ev, _, _ = bufs[b]

        def group(g, cc):
            dvec = dv[pl.ds(g * 16, 16)]

            @pl.when(c == 0)
            def _():
                plsc.addupdate_scatter(cnt, (dvec,), ones16)

            for h in range(8):
                e0 = g * 16 + h * 2
                drep = plsc.load_gather(dv, (pat01 + e0,))
                idx = drep * 8 + iota8x2
                data = plsc.load_gather(ev, (dpat + e0 * 16,))
                plsc.addupdate_scatter(acc, (idx,), data)
            return cc

        lax.fori_loop(0, GRP1, group, 0)

    start(0, 0)

    def pair(p, carry):
        j0 = 2 * p
        start(j0 + 1, 1)
        wait(j0, 0)
        process(0)

        @pl.when(j0 + 2 < NCH1)
        def _():
            start(j0 + 2, 0)

        wait(j0 + 1, 1)
        process(1)
        return carry

    lax.fori_loop(0, NCH1 // 2, pair, 0)

    out_row = c * NS + s
    pltpu.sync_copy(acc, sum_out.at[out_row])

    @pl.when(c == 0)
    def _():
        pltpu.sync_copy(cnt, cnt_out.at[s])


def _segment_sums(dst_r, ef_r, zeros_in):
    f = pl.kernel(
        _seg_body,
        out_type=[
            jax.ShapeDtypeStruct((NW, ACC_W), jnp.float32),
            jax.ShapeDtypeStruct((NS, NPAD), jnp.float32),
        ],
        mesh=_mesh(),
        compiler_params=pltpu.CompilerParams(needs_layout_passes=False),
        scratch_types=[
            pltpu.VMEM((CH1,), jnp.int32),
            pltpu.VMEM((CH1,), jnp.int32),
            pltpu.VMEM((CH1 * 16,), jnp.float32),
            pltpu.VMEM((CH1 * 16,), jnp.float32),
            pltpu.VMEM((ACC_W,), jnp.float32),
            pltpu.VMEM((NPAD,), jnp.float32),
            pltpu.SemaphoreType.DMA,
            pltpu.SemaphoreType.DMA,
            pltpu.SemaphoreType.DMA,
            pltpu.SemaphoreType.DMA,
        ],
    )
    return f(dst_r, ef_r, zeros_in)


def _proj_body(sp, cp, wn, bn, w1, w2, be, h_ref, a_ref, b_ref):
    bm = h_ref.shape[0]
    x = sp[...]
    lo = x[:NS].reshape(NS, bm, 8).sum(axis=0)
    hi = x[NS:].reshape(NS, bm, 8).sum(axis=0)
    sums = jnp.concatenate([lo, hi], axis=1)
    cnts = jnp.maximum(cp[...].sum(axis=0), 1.0)[:, None]
    hn = sums / cnts
    h = jnp.maximum(
        lax.dot_general(hn, wn[...], (((1,), (1,)), ((), ())),
                        preferred_element_type=jnp.float32) + bn[...],
        0.0)
    h_ref[...] = h
    a_ref[...] = lax.dot_general(h, w1[...], (((1,), (1,)), ((), ())),
                                 preferred_element_type=jnp.float32) + be[...]
    b_ref[...] = lax.dot_general(h, w2[...], (((1,), (1,)), ((), ())),
                                 preferred_element_type=jnp.float32)


def _projection(sums_p, cnts_p, W_neigh, b_neigh, W1, W2, b_edge):
    BM = 1024
    grid = NPAD // BM
    full2 = lambda shp: pl.BlockSpec(shp, lambda i: (0, 0))
    outp = pl.BlockSpec((BM, NDIM), lambda i: (i, 0))
    return pl.pallas_call(
        _proj_body,
        grid=(grid,),
        in_specs=[
            pl.BlockSpec((NW, BM * 8), lambda i: (0, i)),
            pl.BlockSpec((NS, BM), lambda i: (0, i)),
            full2((NDIM, EDIM)),
            full2((1, NDIM)),
            full2((NDIM, NDIM)),
            full2((NDIM, NDIM)),
            full2((1, NDIM)),
        ],
        out_specs=[outp, outp, outp],
        out_shape=[
            jax.ShapeDtypeStruct((NPAD, NDIM), jnp.float32),
            jax.ShapeDtypeStruct((NPAD, NDIM), jnp.float32),
            jax.ShapeDtypeStruct((NPAD, NDIM), jnp.float32),
        ],
    )(sums_p, cnts_p, W_neigh, b_neigh.reshape(1, NDIM), W1, W2,
      b_edge.reshape(1, NDIM))


def _pair_body(su_hbm, sv_hbm, a_hbm, b_hbm, out_hbm, idxu_v, idxv_v, bufa, bufb):
    c = lax.axis_index("c")
    s = lax.axis_index("s")
    w = s * NC + c
    base = jnp.where(w < S_SPLIT, w * 3128,
                     S_SPLIT * 3128 + (w - S_SPLIT) * 3120)

    pltpu.sync_copy(su_hbm.at[pl.ds(base, 3072)], idxu_v.at[pl.ds(0, 3072)])
    pltpu.sync_copy(sv_hbm.at[pl.ds(base, 3072)], idxv_v.at[pl.ds(0, 3072)])

    @pl.when(w < S_SPLIT)
    def _():
        pltpu.sync_copy(su_hbm.at[pl.ds(base + 3072, S_TAIL_BIG)],
                        idxu_v.at[pl.ds(3072, S_TAIL_BIG)])
        pltpu.sync_copy(sv_hbm.at[pl.ds(base + 3072, S_TAIL_BIG)],
                        idxv_v.at[pl.ds(3072, S_TAIL_BIG)])

    @pl.when(w >= S_SPLIT)
    def _():
        pltpu.sync_copy(su_hbm.at[pl.ds(base + 3072, S_TAIL_SMALL)],
                        idxu_v.at[pl.ds(3072, S_TAIL_SMALL)])
        pltpu.sync_copy(sv_hbm.at[pl.ds(base + 3072, S_TAIL_SMALL)],
                        idxv_v.at[pl.ds(3072, S_TAIL_SMALL)])

    def do_chunk(j, n):
        off = base + j * S_CHUNK
        pltpu.sync_copy(a_hbm.at[idxu_v.at[pl.ds(j * S_CHUNK, n)]], bufa.at[pl.ds(0, n)])
        pltpu.sync_copy(b_hbm.at[idxv_v.at[pl.ds(j * S_CHUNK, n)]], bufb.at[pl.ds(0, n)])

        def addrow(p, cc):
            for rr in range(2):
                r = p * 2 + rr
                for k in range(NDIM // 16):
                    sl = pl.ds(k * 16, 16)
                    bufa[r, sl] = bufa[r, sl] + bufb[r, sl]
            return cc

        lax.fori_loop(0, n // 2, addrow, 0)
        pltpu.sync_copy(bufa.at[pl.ds(0, n)], out_hbm.at[pl.ds(off, n)])

    def chunk(j, carry):
        do_chunk(j, S_CHUNK)
        return carry

    lax.fori_loop(0, S_NFULL, chunk, 0)

    @pl.when(w < S_SPLIT)
    def _():
        do_chunk(S_NFULL, S_TAIL_BIG)

    @pl.when(w >= S_SPLIT)
    def _():
        do_chunk(S_NFULL, S_TAIL_SMALL)


def _pair_gather(su_r, sv_r, A, B):
    f = pl.kernel(
        _pair_body,
        out_type=[jax.ShapeDtypeStruct((N_SEEDS_K, NDIM), jnp.float32)],
        mesh=_mesh(),
        compiler_params=pltpu.CompilerParams(needs_layout_passes=False),
        scratch_types=[
            pltpu.VMEM((3136,), jnp.int32),
            pltpu.VMEM((3136,), jnp.int32),
            pltpu.VMEM((S_CHUNK, NDIM), jnp.float32),
            pltpu.VMEM((S_CHUNK, NDIM), jnp.float32),
        ],
    )
    return f(su_r, sv_r, A, B)[0]


def kernel(nfeats, efeats, edge_index, seeds_u, seeds_v, W_neigh, b_neigh, W_edge, b_edge):
    del nfeats
    dst_r = edge_index[1].astype(jnp.int32)
    ef_r = efeats.reshape(NS * NCH1, CH1 * 16)
    zeros_in = jnp.zeros((ACC_W,), jnp.float32)
    sums_p, cnts_p = _segment_sums(dst_r, ef_r, zeros_in)
    W1 = W_edge[:, :NDIM]
    W2 = W_edge[:, NDIM:]
    h, A, B = _projection(sums_p, cnts_p, W_neigh, b_neigh, W1, W2, b_edge)
    su_r = seeds_u.astype(jnp.int32)
    sv_r = seeds_v.astype(jnp.int32)
    edge = _pair_gather(su_r, sv_r, A, B)
    return (h[:N_NODES_K], edge)


# speedup vs baseline: 1.3536x; 1.1589x over previous
import jax
import jax.numpy as jnp
from jax import lax
from jax.experimental import pallas as pl
from jax.experimental.pallas import tpu as pltpu
from jax.experimental.pallas import tpu_sc as plsc

N_NODES_K = 10000
NPAD = 10240
N_EDGES_K = 320000
NDIM = 128
EDIM = 16
N_SEEDS_K = 100000

NC = 2
NS = 16
NW = NC * NS

EPT1 = N_EDGES_K // NS
CH1 = 400
NCH1 = EPT1 // CH1
GRP1 = CH1 // 16
ACC_W = NPAD * 8

S_CHUNK = 128
S_NFULL = 24
S_TAIL_BIG = 3128 - S_NFULL * S_CHUNK
S_TAIL_SMALL = 3120 - S_NFULL * S_CHUNK
S_SPLIT = 20

_mesh = lambda: plsc.VectorSubcoreMesh(core_axis_name="c", subcore_axis_name="s")


def _seg_body(dst_hbm, ef_hbm, zeros_hbm, sum_out, cnt_out,
              dst_v0, dst_v1, ef_v0, ef_v1, acc, cnt,
              sd0, sd1, se0, se1):
    c = lax.axis_index("c")
    s = lax.axis_index("s")

    for k in range(ACC_W // 8192):
        pltpu.sync_copy(zeros_hbm.at[pl.ds(k * 8192, 8192)],
                        acc.at[pl.ds(k * 8192, 8192)])
    pltpu.sync_copy(zeros_hbm.at[pl.ds(0, NPAD)], cnt)

    lane = jax.lax.broadcasted_iota(jnp.int32, (16,), 0)
    pat01 = lane // 8
    iota8x2 = lane % 8
    ones16 = jnp.full((16,), 1.0, jnp.float32)

    dpat = pat01 * 16 + iota8x2 + c * 8

    bufs = ((dst_v0, ef_v0, sd0, se0), (dst_v1, ef_v1, sd1, se1))

    def start(j, b):
        dv, ev, sd, se = bufs[b]
        e_base = s * EPT1 + j * CH1
        pltpu.async_copy(dst_hbm.at[pl.ds(e_base, CH1)], dv, sd)
        pltpu.async_copy(ef_hbm.at[s * NCH1 + j], ev, se)

    def wait(j, b):
        dv, ev, sd, se = bufs[b]
        e_base = s * EPT1 + j * CH1
        pltpu.make_async_copy(dst_hbm.at[pl.ds(e_base, CH1)], dv, sd).wait()
        pltpu.make_async_copy(ef_hbm.at[s * NCH1 + j], ev, se).wait()

    def process(b):
        dv, ev, _, _ = bufs[b]

        def group(g, cc):
            dvec = dv[pl.ds(g * 16, 16)]

            @pl.when(c == 0)
            def _():
                plsc.addupdate_scatter(cnt, (dvec,), ones16)

            for h in range(8):
                e0 = g * 16 + h * 2
                drep = plsc.load_gather(dv, (pat01 + e0,))
                idx = drep * 8 + iota8x2
                data = plsc.load_gather(ev, (dpat + e0 * 16,))
                plsc.addupdate_scatter(acc, (idx,), data)
            return cc

        lax.fori_loop(0, GRP1, group, 0)

    start(0, 0)

    def pair(p, carry):
        j0 = 2 * p
        start(j0 + 1, 1)
        wait(j0, 0)
        process(0)

        @pl.when(j0 + 2 < NCH1)
        def _():
            start(j0 + 2, 0)

        wait(j0 + 1, 1)
        process(1)
        return carry

    lax.fori_loop(0, NCH1 // 2, pair, 0)

    out_row = c * NS + s
    pltpu.sync_copy(acc, sum_out.at[out_row])

    @pl.when(c == 0)
    def _():
        pltpu.sync_copy(cnt, cnt_out.at[s])


def _segment_sums(dst_r, ef_r, zeros_in):
    f = pl.kernel(
        _seg_body,
        out_type=[
            jax.ShapeDtypeStruct((NW, ACC_W), jnp.float32),
            jax.ShapeDtypeStruct((NS, NPAD), jnp.float32),
        ],
        mesh=_mesh(),
        compiler_params=pltpu.CompilerParams(needs_layout_passes=False),
        scratch_types=[
            pltpu.VMEM((CH1,), jnp.int32),
            pltpu.VMEM((CH1,), jnp.int32),
            pltpu.VMEM((CH1 * 16,), jnp.float32),
            pltpu.VMEM((CH1 * 16,), jnp.float32),
            pltpu.VMEM((ACC_W,), jnp.float32),
            pltpu.VMEM((NPAD,), jnp.float32),
            pltpu.SemaphoreType.DMA,
            pltpu.SemaphoreType.DMA,
            pltpu.SemaphoreType.DMA,
            pltpu.SemaphoreType.DMA,
        ],
    )
    return f(dst_r, ef_r, zeros_in)


def _proj_body(sp, cp, wn, bn, w1, w2, be, h_ref, a_ref, b_ref):
    bm = h_ref.shape[0]
    x = sp[...]
    lo = x[:NS].reshape(NS, bm, 8).sum(axis=0)
    hi = x[NS:].reshape(NS, bm, 8).sum(axis=0)
    sums = jnp.concatenate([lo, hi], axis=1)
    cnts = jnp.maximum(cp[...].sum(axis=0), 1.0)[:, None]
    hn = sums / cnts
    h = jnp.maximum(
        lax.dot_general(hn, wn[...], (((1,), (1,)), ((), ())),
                        preferred_element_type=jnp.float32) + bn[...],
        0.0)
    h_ref[...] = h
    a_ref[...] = lax.dot_general(h, w1[...], (((1,), (1,)), ((), ())),
                                 preferred_element_type=jnp.float32) + be[...]
    b_ref[...] = lax.dot_general(h, w2[...], (((1,), (1,)), ((), ())),
                                 preferred_element_type=jnp.float32)


def _projection(sums_p, cnts_p, W_neigh, b_neigh, W1, W2, b_edge):
    BM = 1024
    grid = NPAD // BM
    full2 = lambda shp: pl.BlockSpec(shp, lambda i: (0, 0))
    outp = pl.BlockSpec((BM, NDIM), lambda i: (i, 0))
    return pl.pallas_call(
        _proj_body,
        grid=(grid,),
        in_specs=[
            pl.BlockSpec((NW, BM * 8), lambda i: (0, i)),
            pl.BlockSpec((NS, BM), lambda i: (0, i)),
            full2((NDIM, EDIM)),
            full2((1, NDIM)),
            full2((NDIM, NDIM)),
            full2((NDIM, NDIM)),
            full2((1, NDIM)),
        ],
        out_specs=[outp, outp, outp],
        out_shape=[
            jax.ShapeDtypeStruct((NPAD, NDIM), jnp.float32),
            jax.ShapeDtypeStruct((NPAD, NDIM), jnp.float32),
            jax.ShapeDtypeStruct((NPAD, NDIM), jnp.float32),
        ],
    )(sums_p, cnts_p, W_neigh, b_neigh.reshape(1, NDIM), W1, W2,
      b_edge.reshape(1, NDIM))


def _pair_body(su_hbm, sv_hbm, a_hbm, b_hbm, out_hbm, idxu_v, idxv_v,
               bufa0, bufb0, bufa1, bufb1, sa0, sb0, sa1, sb1):
    c = lax.axis_index("c")
    s = lax.axis_index("s")
    w = s * NC + c
    base = jnp.where(w < S_SPLIT, w * 3128,
                     S_SPLIT * 3128 + (w - S_SPLIT) * 3120)

    pltpu.sync_copy(su_hbm.at[pl.ds(base, 3072)], idxu_v.at[pl.ds(0, 3072)])
    pltpu.sync_copy(sv_hbm.at[pl.ds(base, 3072)], idxv_v.at[pl.ds(0, 3072)])

    @pl.when(w < S_SPLIT)
    def _():
        pltpu.sync_copy(su_hbm.at[pl.ds(base + 3072, S_TAIL_BIG)],
                        idxu_v.at[pl.ds(3072, S_TAIL_BIG)])
        pltpu.sync_copy(sv_hbm.at[pl.ds(base + 3072, S_TAIL_BIG)],
                        idxv_v.at[pl.ds(3072, S_TAIL_BIG)])

    @pl.when(w >= S_SPLIT)
    def _():
        pltpu.sync_copy(su_hbm.at[pl.ds(base + 3072, S_TAIL_SMALL)],
                        idxu_v.at[pl.ds(3072, S_TAIL_SMALL)])
        pltpu.sync_copy(sv_hbm.at[pl.ds(base + 3072, S_TAIL_SMALL)],
                        idxv_v.at[pl.ds(3072, S_TAIL_SMALL)])

    gbufs = ((bufa0, bufb0, sa0, sb0), (bufa1, bufb1, sa1, sb1))

    def gstart(j, p):
        ba, bb, sa, sb = gbufs[p]
        pltpu.async_copy(a_hbm.at[idxu_v.at[pl.ds(j * S_CHUNK, S_CHUNK)]], ba, sa)
        pltpu.async_copy(b_hbm.at[idxv_v.at[pl.ds(j * S_CHUNK, S_CHUNK)]], bb, sb)

    def gwait(j, p):
        ba, bb, sa, sb = gbufs[p]
        pltpu.make_async_copy(a_hbm.at[idxu_v.at[pl.ds(j * S_CHUNK, S_CHUNK)]], ba, sa).wait()
        pltpu.make_async_copy(b_hbm.at[idxv_v.at[pl.ds(j * S_CHUNK, S_CHUNK)]], bb, sb).wait()

    def addwrite(j, p, n):
        ba, bb, _, _ = gbufs[p]

        def addrow(q, cc):
            for rr in range(2):
                r = q * 2 + rr
                for k in range(NDIM // 16):
                    sl = pl.ds(k * 16, 16)
                    ba[r, sl] = ba[r, sl] + bb[r, sl]
            return cc

        lax.fori_loop(0, n // 2, addrow, 0)
        pltpu.sync_copy(ba.at[pl.ds(0, n)], out_hbm.at[pl.ds(base + j * S_CHUNK, n)])

    def do_tail(n):
        ba, bb, sa, sb = gbufs[0]
        pltpu.async_copy(a_hbm.at[idxu_v.at[pl.ds(S_NFULL * S_CHUNK, n)]], ba.at[pl.ds(0, n)], sa)
        pltpu.async_copy(b_hbm.at[idxv_v.at[pl.ds(S_NFULL * S_CHUNK, n)]], bb.at[pl.ds(0, n)], sb)
        pltpu.make_async_copy(a_hbm.at[idxu_v.at[pl.ds(S_NFULL * S_CHUNK, n)]], ba.at[pl.ds(0, n)], sa).wait()
        pltpu.make_async_copy(b_hbm.at[idxv_v.at[pl.ds(S_NFULL * S_CHUNK, n)]], bb.at[pl.ds(0, n)], sb).wait()
        addwrite(S_NFULL, 0, n)

    gstart(0, 0)

    def pair(p, carry):
        j0 = 2 * p
        gstart(j0 + 1, 1)
        gwait(j0, 0)
        addwrite(j0, 0, S_CHUNK)

        @pl.when(j0 + 2 < S_NFULL)
        def _():
            gstart(j0 + 2, 0)

        gwait(j0 + 1, 1)
        addwrite(j0 + 1, 1, S_CHUNK)
        return carry

    lax.fori_loop(0, S_NFULL // 2, pair, 0)

    @pl.when(w < S_SPLIT)
    def _():
        do_tail(S_TAIL_BIG)

    @pl.when(w >= S_SPLIT)
    def _():
        do_tail(S_TAIL_SMALL)


def _pair_gather(su_r, sv_r, A, B):
    f = pl.kernel(
        _pair_body,
        out_type=[jax.ShapeDtypeStruct((N_SEEDS_K, NDIM), jnp.float32)],
        mesh=_mesh(),
        compiler_params=pltpu.CompilerParams(needs_layout_passes=False),
        scratch_types=[
            pltpu.VMEM((3136,), jnp.int32),
            pltpu.VMEM((3136,), jnp.int32),
            pltpu.VMEM((S_CHUNK, NDIM), jnp.float32),
            pltpu.VMEM((S_CHUNK, NDIM), jnp.float32),
            pltpu.VMEM((S_CHUNK, NDIM), jnp.float32),
            pltpu.VMEM((S_CHUNK, NDIM), jnp.float32),
            pltpu.SemaphoreType.DMA,
            pltpu.SemaphoreType.DMA,
            pltpu.SemaphoreType.DMA,
            pltpu.SemaphoreType.DMA,
        ],
    )
    return f(su_r, sv_r, A, B)[0]


def kernel(nfeats, efeats, edge_index, seeds_u, seeds_v, W_neigh, b_neigh, W_edge, b_edge):
    del nfeats
    dst_r = edge_index[1].astype(jnp.int32)
    ef_r = efeats.reshape(NS * NCH1, CH1 * 16)
    zeros_in = jnp.zeros((ACC_W,), jnp.float32)
    sums_p, cnts_p = _segment_sums(dst_r, ef_r, zeros_in)
    W1 = W_edge[:, :NDIM]
    W2 = W_edge[:, NDIM:]
    h, A, B = _projection(sums_p, cnts_p, W_neigh, b_neigh, W1, W2, b_edge)
    su_r = seeds_u.astype(jnp.int32)
    sv_r = seeds_v.astype(jnp.int32)
    edge = _pair_gather(su_r, sv_r, A, B)
    return (h[:N_NODES_K], edge)


# speedup vs baseline: 1.3558x; 1.0017x over previous
import jax
import jax.numpy as jnp
from jax import lax
from jax.experimental import pallas as pl
from jax.experimental.pallas import tpu as pltpu
from jax.experimental.pallas import tpu_sc as plsc

N_NODES_K = 10000
NPAD = 10240
N_EDGES_K = 320000
NDIM = 128
EDIM = 16
N_SEEDS_K = 100000

NC = 2
NS = 16
NW = NC * NS

EPT1 = N_EDGES_K // NS
CH1 = 400
NCH1 = EPT1 // CH1
GRP1 = CH1 // 16
ACC_W = NPAD * 8

S_CHUNK = 128
S_NFULL = 24
S_TAIL_BIG = 3128 - S_NFULL * S_CHUNK
S_TAIL_SMALL = 3120 - S_NFULL * S_CHUNK
S_SPLIT = 20

_mesh = lambda: plsc.VectorSubcoreMesh(core_axis_name="c", subcore_axis_name="s")


def _seg_body(dst_hbm, ef_hbm, zeros_hbm, sum_out, cnt_out,
              dst_v0, dst_v1, ef_v0, ef_v1, acc, cnt,
              sd0, sd1, se0, se1):
    c = lax.axis_index("c")
    s = lax.axis_index("s")

    for k in range(ACC_W // 8192):
        pltpu.sync_copy(zeros_hbm.at[pl.ds(k * 8192, 8192)],
                        acc.at[pl.ds(k * 8192, 8192)])
    pltpu.sync_copy(zeros_hbm.at[pl.ds(0, NPAD)], cnt)

    lane = jax.lax.broadcasted_iota(jnp.int32, (16,), 0)
    pat01 = lane // 8
    iota8x2 = lane % 8
    ones16 = jnp.full((16,), 1.0, jnp.float32)

    dpat = pat01 * 16 + iota8x2 + c * 8

    bufs = ((dst_v0, ef_v0, sd0, se0), (dst_v1, ef_v1, sd1, se1))

    def start(j, b):
        dv, ev, sd, se = bufs[b]
        e_base = s * EPT1 + j * CH1
        pltpu.async_copy(dst_hbm.at[pl.ds(e_base, CH1)], dv, sd)
        pltpu.async_copy(ef_hbm.at[s * NCH1 + j], ev, se)

    def wait(j, b):
        dv, ev, sd, se = bufs[b]
        e_base = s * EPT1 + j * CH1
        pltpu.make_async_copy(dst_hbm.at[pl.ds(e_base, CH1)], dv, sd).wait()
        pltpu.make_async_copy(ef_hbm.at[s * NCH1 + j], ev, se).wait()

    def process(b):
        dv, ev, _, _ = bufs[b]

        def group(g, cc):
            dvec = dv[pl.ds(g * 16, 16)]

            @pl.when(c == 0)
            def _():
                plsc.addupdate_scatter(cnt, (dvec,), ones16)

            for h in range(8):
                e0 = g * 16 + h * 2
                drep = plsc.load_gather(dv, (pat01 + e0,))
                idx = drep * 8 + iota8x2
                data = plsc.load_gather(ev, (dpat + e0 * 16,))
                plsc.addupdate_scatter(acc, (idx,), data)
            return cc

        lax.fori_loop(0, GRP1, group, 0)

    start(0, 0)

    def pair(p, carry):
        j0 = 2 * p
        start(j0 + 1, 1)
        wait(j0, 0)
        process(0)

        @pl.when(j0 + 2 < NCH1)
        def _():
            start(j0 + 2, 0)

        wait(j0 + 1, 1)
        process(1)
        return carry

    lax.fori_loop(0, NCH1 // 2, pair, 0)

    out_row = c * NS + s
    pltpu.sync_copy(acc, sum_out.at[out_row])

    @pl.when(c == 0)
    def _():
        pltpu.sync_copy(cnt, cnt_out.at[s])


def _segment_sums(dst_r, ef_r, zeros_in):
    f = pl.kernel(
        _seg_body,
        out_type=[
            jax.ShapeDtypeStruct((NW, ACC_W), jnp.float32),
            jax.ShapeDtypeStruct((NS, NPAD), jnp.float32),
        ],
        mesh=_mesh(),
        compiler_params=pltpu.CompilerParams(needs_layout_passes=False),
        scratch_types=[
            pltpu.VMEM((CH1,), jnp.int32),
            pltpu.VMEM((CH1,), jnp.int32),
            pltpu.VMEM((CH1 * 16,), jnp.float32),
            pltpu.VMEM((CH1 * 16,), jnp.float32),
            pltpu.VMEM((ACC_W,), jnp.float32),
            pltpu.VMEM((NPAD,), jnp.float32),
            pltpu.SemaphoreType.DMA,
            pltpu.SemaphoreType.DMA,
            pltpu.SemaphoreType.DMA,
            pltpu.SemaphoreType.DMA,
        ],
    )
    return f(dst_r, ef_r, zeros_in)


def _proj_body(sp, cp, wn, bn, w1, w2, be, h_ref, a_ref, b_ref):
    bm = h_ref.shape[0]
    x = sp[...]
    lo = x[:NS].reshape(NS, bm, 8).sum(axis=0)
    hi = x[NS:].reshape(NS, bm, 8).sum(axis=0)
    sums = jnp.concatenate([lo, hi], axis=1)
    cnts = jnp.maximum(cp[...].sum(axis=0), 1.0)[:, None]
    hn = sums / cnts
    h = jnp.maximum(
        lax.dot_general(hn, wn[...], (((1,), (1,)), ((), ())),
                        preferred_element_type=jnp.float32) + bn[...],
        0.0)
    h_ref[...] = h
    a_ref[...] = lax.dot_general(h, w1[...], (((1,), (1,)), ((), ())),
                                 preferred_element_type=jnp.float32) + be[...]
    b_ref[...] = lax.dot_general(h, w2[...], (((1,), (1,)), ((), ())),
                                 preferred_element_type=jnp.float32)


def _projection(sums_p, cnts_p, W_neigh, b_neigh, W1, W2, b_edge):
    BM = 2048
    grid = NPAD // BM
    full2 = lambda shp: pl.BlockSpec(shp, lambda i: (0, 0))
    outp = pl.BlockSpec((BM, NDIM), lambda i: (i, 0))
    return pl.pallas_call(
        _proj_body,
        grid=(grid,),
        in_specs=[
            pl.BlockSpec((NW, BM * 8), lambda i: (0, i)),
            pl.BlockSpec((NS, BM), lambda i: (0, i)),
            full2((NDIM, EDIM)),
            full2((1, NDIM)),
            full2((NDIM, NDIM)),
            full2((NDIM, NDIM)),
            full2((1, NDIM)),
        ],
        out_specs=[outp, outp, outp],
        out_shape=[
            jax.ShapeDtypeStruct((NPAD, NDIM), jnp.float32),
            jax.ShapeDtypeStruct((NPAD, NDIM), jnp.float32),
            jax.ShapeDtypeStruct((NPAD, NDIM), jnp.float32),
        ],
    )(sums_p, cnts_p, W_neigh, b_neigh.reshape(1, NDIM), W1, W2,
      b_edge.reshape(1, NDIM))


def _pair_body(su_hbm, sv_hbm, a_hbm, b_hbm, out_hbm, idxu_v, idxv_v,
               bufa0, bufb0, bufa1, bufb1, sa0, sb0, sa1, sb1):
    c = lax.axis_index("c")
    s = lax.axis_index("s")
    w = s * NC + c
    base = jnp.where(w < S_SPLIT, w * 3128,
                     S_SPLIT * 3128 + (w - S_SPLIT) * 3120)

    pltpu.sync_copy(su_hbm.at[pl.ds(base, 3072)], idxu_v.at[pl.ds(0, 3072)])
    pltpu.sync_copy(sv_hbm.at[pl.ds(base, 3072)], idxv_v.at[pl.ds(0, 3072)])

    @pl.when(w < S_SPLIT)
    def _():
        pltpu.sync_copy(su_hbm.at[pl.ds(base + 3072, S_TAIL_BIG)],
                        idxu_v.at[pl.ds(3072, S_TAIL_BIG)])
        pltpu.sync_copy(sv_hbm.at[pl.ds(base + 3072, S_TAIL_BIG)],
                        idxv_v.at[pl.ds(3072, S_TAIL_BIG)])

    @pl.when(w >= S_SPLIT)
    def _():
        pltpu.sync_copy(su_hbm.at[pl.ds(base + 3072, S_TAIL_SMALL)],
                        idxu_v.at[pl.ds(3072, S_TAIL_SMALL)])
        pltpu.sync_copy(sv_hbm.at[pl.ds(base + 3072, S_TAIL_SMALL)],
                        idxv_v.at[pl.ds(3072, S_TAIL_SMALL)])

    gbufs = ((bufa0, bufb0, sa0, sb0), (bufa1, bufb1, sa1, sb1))

    def gstart(j, p):
        ba, bb, sa, sb = gbufs[p]
        pltpu.async_copy(a_hbm.at[idxu_v.at[pl.ds(j * S_CHUNK, S_CHUNK)]], ba, sa)
        pltpu.async_copy(b_hbm.at[idxv_v.at[pl.ds(j * S_CHUNK, S_CHUNK)]], bb, sb)

    def gwait(j, p):
        ba, bb, sa, sb = gbufs[p]
        pltpu.make_async_copy(a_hbm.at[idxu_v.at[pl.ds(j * S_CHUNK, S_CHUNK)]], ba, sa).wait()
        pltpu.make_async_copy(b_hbm.at[idxv_v.at[pl.ds(j * S_CHUNK, S_CHUNK)]], bb, sb).wait()

    def addwrite(j, p, n):
        ba, bb, _, _ = gbufs[p]

        def addrow(q, cc):
            for rr in range(2):
                r = q * 2 + rr
                for k in range(NDIM // 16):
                    sl = pl.ds(k * 16, 16)
                    ba[r, sl] = ba[r, sl] + bb[r, sl]
            return cc

        lax.fori_loop(0, n // 2, addrow, 0)
        pltpu.sync_copy(ba.at[pl.ds(0, n)], out_hbm.at[pl.ds(base + j * S_CHUNK, n)])

    def do_tail(n):
        ba, bb, sa, sb = gbufs[0]
        pltpu.async_copy(a_hbm.at[idxu_v.at[pl.ds(S_NFULL * S_CHUNK, n)]], ba.at[pl.ds(0, n)], sa)
        pltpu.async_copy(b_hbm.at[idxv_v.at[pl.ds(S_NFULL * S_CHUNK, n)]], bb.at[pl.ds(0, n)], sb)
        pltpu.make_async_copy(a_hbm.at[idxu_v.at[pl.ds(S_NFULL * S_CHUNK, n)]], ba.at[pl.ds(0, n)], sa).wait()
        pltpu.make_async_copy(b_hbm.at[idxv_v.at[pl.ds(S_NFULL * S_CHUNK, n)]], bb.at[pl.ds(0, n)], sb).wait()
        addwrite(S_NFULL, 0, n)

    gstart(0, 0)

    def pair(p, carry):
        j0 = 2 * p
        gstart(j0 + 1, 1)
        gwait(j0, 0)
        addwrite(j0, 0, S_CHUNK)

        @pl.when(j0 + 2 < S_NFULL)
        def _():
            gstart(j0 + 2, 0)

        gwait(j0 + 1, 1)
        addwrite(j0 + 1, 1, S_CHUNK)
        return carry

    lax.fori_loop(0, S_NFULL // 2, pair, 0)

    @pl.when(w < S_SPLIT)
    def _():
        do_tail(S_TAIL_BIG)

    @pl.when(w >= S_SPLIT)
    def _():
        do_tail(S_TAIL_SMALL)


def _pair_gather(su_r, sv_r, A, B):
    f = pl.kernel(
        _pair_body,
        out_type=[jax.ShapeDtypeStruct((N_SEEDS_K, NDIM), jnp.float32)],
        mesh=_mesh(),
        compiler_params=pltpu.CompilerParams(needs_layout_passes=False),
        scratch_types=[
            pltpu.VMEM((3136,), jnp.int32),
            pltpu.VMEM((3136,), jnp.int32),
            pltpu.VMEM((S_CHUNK, NDIM), jnp.float32),
            pltpu.VMEM((S_CHUNK, NDIM), jnp.float32),
            pltpu.VMEM((S_CHUNK, NDIM), jnp.float32),
            pltpu.VMEM((S_CHUNK, NDIM), jnp.float32),
            pltpu.SemaphoreType.DMA,
            pltpu.SemaphoreType.DMA,
            pltpu.SemaphoreType.DMA,
            pltpu.SemaphoreType.DMA,
        ],
    )
    return f(su_r, sv_r, A, B)[0]


def kernel(nfeats, efeats, edge_index, seeds_u, seeds_v, W_neigh, b_neigh, W_edge, b_edge):
    del nfeats
    dst_r = edge_index[1].astype(jnp.int32)
    ef_r = efeats.reshape(NS * NCH1, CH1 * 16)
    zeros_in = jnp.zeros((ACC_W,), jnp.float32)
    sums_p, cnts_p = _segment_sums(dst_r, ef_r, zeros_in)
    W1 = W_edge[:, :NDIM]
    W2 = W_edge[:, NDIM:]
    h, A, B = _projection(sums_p, cnts_p, W_neigh, b_neigh, W1, W2, b_edge)
    su_r = seeds_u.astype(jnp.int32)
    sv_r = seeds_v.astype(jnp.int32)
    edge = _pair_gather(su_r, sv_r, A, B)
    return (h[:N_NODES_K], edge)
